# Initial kernel scaffold; baseline (speedup 1.0000x reference)
import functools

import jax
import jax.numpy as jnp
from jax import lax
from jax.experimental import pallas as pl
from jax.experimental.pallas import tpu as pltpu
from jax.experimental.pallas import tpu_sc as plsc

NC = 2
NS = 16
NW = NC * NS
LANES = 16
CHUNK = 128


def _round_up(a, m):
    return (a + m - 1) // m * m


def _deg_kernel(dst_pad, n_pad, per_w):
    ch = per_w // CHUNK
    mesh = plsc.VectorSubcoreMesh(core_axis_name="c", subcore_axis_name="s")

    @functools.partial(
        pl.kernel,
        out_type=jax.ShapeDtypeStruct((NW, n_pad), jnp.float32),
        mesh=mesh,
        scratch_types=[
            pltpu.VMEM((CHUNK,), jnp.int32),
            pltpu.VMEM((n_pad,), jnp.float32),
        ],
    )
    def k(dst_hbm, out_hbm, didx, degbuf):
        w = lax.axis_index("c") * NS + lax.axis_index("s")
        zeros = jnp.zeros((LANES,), jnp.float32)
        ones = jnp.ones((LANES,), jnp.float32)

        @pl.loop(0, n_pad, step=LANES)
        def _(i):
            degbuf[pl.ds(i, LANES)] = zeros

        base = w * per_w

        @pl.loop(0, ch)
        def _(c):
            pltpu.sync_copy(dst_hbm.at[pl.ds(base + c * CHUNK, CHUNK)], didx)
            for j in range(CHUNK // LANES):
                idx = didx[pl.ds(j * LANES, LANES)]
                plsc.addupdate_scatter(degbuf, [idx], ones)

        pltpu.sync_copy(degbuf, out_hbm.at[w])

    return k(dst_pad)


def _prescale_kernel(deg_parts, x, n):
    def body(degp_ref, x_ref, xt_ref):
        degp = degp_ref[:, :n]
        ones = jnp.ones((NW, 1), jnp.float32)
        deg = lax.dot_general(
            degp, ones, (((0,), (0,)), ((), ())),
            preferred_element_type=jnp.float32,
            precision=lax.Precision.HIGHEST,
        )
        dis = lax.rsqrt(deg + 1.0)
        xt_ref[...] = x_ref[...] * dis

    return pl.pallas_call(
        body,
        out_shape=jax.ShapeDtypeStruct(x.shape, jnp.float32),
    )(deg_parts, x)


def _propagate_kernel(xt, src_pad, dst_pad, n, n_pad, per_w):
    d = xt.shape[1]
    ch = per_w // CHUNK
    zero_copies = n_pad // NS // CHUNK
    out_rows = n // NS
    mesh = plsc.VectorSubcoreMesh(core_axis_name="c", subcore_axis_name="s")

    @functools.partial(
        pl.kernel,
        out_type=jax.ShapeDtypeStruct((NC, n, d), jnp.float32),
        mesh=mesh,
        scratch_types=[
            pltpu.VMEM((CHUNK,), jnp.int32),
            pltpu.VMEM((1, CHUNK), jnp.int32),
            pltpu.VMEM((CHUNK, d), jnp.float32),
            pltpu.VMEM_SHARED((n_pad, d), jnp.float32),
        ],
    )
    def k(xt_hbm, src_hbm, dst_hbm, out_hbm, sidx, didx, rows, h_sh):
        cid = lax.axis_index("c")
        sid = lax.axis_index("s")
        w = cid * NS + sid
        zeros = jnp.zeros((LANES,), jnp.float32)

        @pl.loop(0, CHUNK)
        def _(r):
            for j in range(d // LANES):
                rows[r, pl.ds(j * LANES, LANES)] = zeros

        @pl.loop(0, zero_copies)
        def _(z):
            pltpu.sync_copy(
                rows, h_sh.at[pl.ds((sid * zero_copies + z) * CHUNK, CHUNK)]
            )

        plsc.subcore_barrier()

        base = w * per_w

        @pl.loop(0, ch)
        def _(c):
            off = base + c * CHUNK
            pltpu.sync_copy(src_hbm.at[pl.ds(off, CHUNK)], sidx)
            pltpu.sync_copy(dst_hbm.at[pl.ds(off, CHUNK)], didx.at[0])
            pltpu.sync_copy(xt_hbm.at[sidx], rows)
            pltpu.sync_copy(rows, h_sh.at[didx.at[0]], add=True)

        plsc.subcore_barrier()

        pltpu.sync_copy(
            h_sh.at[pl.ds(sid * out_rows, out_rows)],
            out_hbm.at[cid, pl.ds(sid * out_rows, out_rows)],
        )

    return k(xt, src_pad, dst_pad)


def _final_kernel(deg_parts, xt, s, w_mat, b_row, n):
    def body(degp_ref, xt_ref, s_ref, w_ref, b_ref, out_ref):
        degp = degp_ref[:, :n]
        ones = jnp.ones((NW, 1), jnp.float32)
        deg = lax.dot_general(
            degp, ones, (((0,), (0,)), ((), ())),
            preferred_element_type=jnp.float32,
            precision=lax.Precision.HIGHEST,
        )
        dis = lax.rsqrt(deg + 1.0)
        h = (s_ref[0] + s_ref[1] + xt_ref[...]) * dis
        y = lax.dot_general(
            h, w_ref[...], (((1,), (1,)), ((), ())),
            preferred_element_type=jnp.float32,
            precision=lax.Precision.HIGHEST,
        ) + b_ref[...]
        out_ref[...] = jnp.where(y >= 0.0, y, 0.1 * y)

    return pl.pallas_call(
        body,
        out_shape=jax.ShapeDtypeStruct(xt.shape, jnp.float32),
    )(deg_parts, xt, s, w_mat, b_row)


def kernel(x, edge_index, W, b):
    n, d = x.shape
    e = edge_index.shape[1]
    e_pad = _round_up(e, NW * CHUNK)
    per_w = e_pad // NW
    n_pad = _round_up(n + 1, NS * CHUNK)

    src = edge_index[0]
    dst = edge_index[1]
    pad = e_pad - e
    src_p = jnp.concatenate([src, jnp.zeros((pad,), jnp.int32)])
    dst_p = jnp.concatenate([dst, jnp.full((pad,), n, jnp.int32)])

    deg_parts = _deg_kernel(dst_p, n_pad, per_w)
    xt = _prescale_kernel(deg_parts, x, n)
    s = _propagate_kernel(xt, src_p, dst_p, n, n_pad, per_w)
    return _final_kernel(deg_parts, xt, s, W, b.reshape(1, d), n)


# speedup vs baseline: 15.2649x; 15.2649x over previous
import dataclasses
import functools

import jax
import jax.numpy as jnp
from jax import lax
from jax.experimental import pallas as pl
from jax.experimental.pallas import tpu as pltpu
from jax.experimental.pallas import tpu_sc as plsc

NC = 2
NS = 16
NW = NC * NS
LANES = 16
CHUNK = 128


def _round_up(a, m):
    return (a + m - 1) // m * m


def _sc_compiler_params():
    cp = pltpu.CompilerParams()
    if "needs_layout_passes" in pltpu.CompilerParams.__dataclass_fields__:
        cp = dataclasses.replace(cp, needs_layout_passes=False)
    return cp


def _deg_kernel(dst_pad, n_pad, per_w):
    ch = per_w // CHUNK
    mesh = plsc.VectorSubcoreMesh(core_axis_name="c", subcore_axis_name="s")

    @functools.partial(
        pl.kernel,
        out_type=jax.ShapeDtypeStruct((NW, n_pad), jnp.float32),
        mesh=mesh,
        scratch_types=[
            pltpu.VMEM((CHUNK,), jnp.int32),
            pltpu.VMEM((n_pad,), jnp.float32),
        ],
        compiler_params=_sc_compiler_params(),
    )
    def k(dst_hbm, out_hbm, didx, degbuf):
        w = lax.axis_index("c") * NS + lax.axis_index("s")
        zeros = jnp.zeros((LANES,), jnp.float32)
        ones = jnp.ones((LANES,), jnp.float32)

        @pl.loop(0, n_pad, step=LANES)
        def _(i):
            degbuf[pl.ds(i, LANES)] = zeros

        base = w * per_w

        @pl.loop(0, ch)
        def _(c):
            pltpu.sync_copy(dst_hbm.at[pl.ds(base + c * CHUNK, CHUNK)], didx)
            for j in range(CHUNK // LANES):
                idx = didx[pl.ds(j * LANES, LANES)]
                plsc.addupdate_scatter(degbuf, [idx], ones)

        pltpu.sync_copy(degbuf, out_hbm.at[w])

    return k(dst_pad)


def _prescale_kernel(deg_parts, x, n):
    def body(degp_ref, x_ref, xt_ref):
        degp = degp_ref[:, :n]
        ones = jnp.ones((NW, 1), jnp.float32)
        deg = lax.dot_general(
            degp, ones, (((0,), (0,)), ((), ())),
            preferred_element_type=jnp.float32,
            precision=lax.Precision.HIGHEST,
        )
        dis = lax.rsqrt(deg + 1.0)
        xt_ref[...] = x_ref[...] * dis

    return pl.pallas_call(
        body,
        out_shape=jax.ShapeDtypeStruct(x.shape, jnp.float32),
    )(deg_parts, x)


def _propagate_kernel(xt, src_pad, dst_pad, n, n_pad, per_w):
    d = xt.shape[1]
    ch = per_w // CHUNK
    zero_copies = n_pad // NS // CHUNK
    out_rows = (n // NS) // 8 * 8
    out_rows_last = n - (NS - 1) * out_rows
    mesh = plsc.VectorSubcoreMesh(core_axis_name="c", subcore_axis_name="s")

    @functools.partial(
        pl.kernel,
        out_type=jax.ShapeDtypeStruct((NC, n, d), jnp.float32),
        mesh=mesh,
        scratch_types=[
            pltpu.VMEM((CHUNK,), jnp.int32),
            pltpu.VMEM((1, CHUNK), jnp.int32),
            pltpu.VMEM((CHUNK, d), jnp.float32),
            pltpu.VMEM_SHARED((n_pad, d), jnp.float32),
        ],
        compiler_params=_sc_compiler_params(),
    )
    def k(xt_hbm, src_hbm, dst_hbm, out_hbm, sidx, didx, rows, h_sh):
        cid = lax.axis_index("c")
        sid = lax.axis_index("s")
        w = cid * NS + sid
        zeros = jnp.zeros((LANES,), jnp.float32)

        @pl.loop(0, CHUNK)
        def _(r):
            for j in range(d // LANES):
                rows[r, pl.ds(j * LANES, LANES)] = zeros

        @pl.loop(0, zero_copies)
        def _(z):
            pltpu.sync_copy(
                rows, h_sh.at[pl.ds((sid * zero_copies + z) * CHUNK, CHUNK)]
            )

        plsc.subcore_barrier()

        base = w * per_w

        @pl.loop(0, ch)
        def _(c):
            off = base + c * CHUNK
            pltpu.sync_copy(src_hbm.at[pl.ds(off, CHUNK)], sidx)
            pltpu.sync_copy(dst_hbm.at[pl.ds(off, CHUNK)], didx.at[0])
            pltpu.sync_copy(xt_hbm.at[sidx], rows)
            pltpu.sync_copy(rows, h_sh.at[didx.at[0]], add=True)

        plsc.subcore_barrier()

        @pl.when(sid < NS - 1)
        def _():
            pltpu.sync_copy(
                h_sh.at[pl.ds(sid * out_rows, out_rows)],
                out_hbm.at[cid, pl.ds(sid * out_rows, out_rows)],
            )

        @pl.when(sid == NS - 1)
        def _():
            pltpu.sync_copy(
                h_sh.at[pl.ds((NS - 1) * out_rows, out_rows_last)],
                out_hbm.at[cid, pl.ds((NS - 1) * out_rows, out_rows_last)],
            )

    return k(xt, src_pad, dst_pad)


def _final_kernel(deg_parts, xt, s, w_mat, b_row, n):
    def body(degp_ref, xt_ref, s_ref, w_ref, b_ref, out_ref):
        degp = degp_ref[:, :n]
        ones = jnp.ones((NW, 1), jnp.float32)
        deg = lax.dot_general(
            degp, ones, (((0,), (0,)), ((), ())),
            preferred_element_type=jnp.float32,
            precision=lax.Precision.HIGHEST,
        )
        dis = lax.rsqrt(deg + 1.0)
        h = (s_ref[0] + s_ref[1] + xt_ref[...]) * dis
        y = lax.dot_general(
            h, w_ref[...], (((1,), (1,)), ((), ())),
            preferred_element_type=jnp.float32,
            precision=lax.Precision.HIGHEST,
        ) + b_ref[...]
        out_ref[...] = jnp.where(y >= 0.0, y, 0.1 * y)

    return pl.pallas_call(
        body,
        out_shape=jax.ShapeDtypeStruct(xt.shape, jnp.float32),
    )(deg_parts, xt, s, w_mat, b_row)


def kernel(x, edge_index, W, b):
    n, d = x.shape
    e = edge_index.shape[1]
    e_pad = _round_up(e, NW * CHUNK)
    per_w = e_pad // NW
    n_pad = _round_up(n + 1, NS * CHUNK)

    src = edge_index[0]
    dst = edge_index[1]
    pad = e_pad - e
    src_p = jnp.concatenate([src, jnp.zeros((pad,), jnp.int32)])
    dst_p = jnp.concatenate([dst, jnp.full((pad,), n, jnp.int32)])

    deg_parts = _deg_kernel(dst_p, n_pad, per_w)
    xt = _prescale_kernel(deg_parts, x, n)
    s = _propagate_kernel(xt, src_p, dst_p, n, n_pad, per_w)
    return _final_kernel(deg_parts, xt, s, W, b.reshape(1, d), n)


# speedup vs baseline: 16.6906x; 1.0934x over previous
import dataclasses
import functools

import jax
import jax.numpy as jnp
from jax import lax
from jax.experimental import pallas as pl
from jax.experimental.pallas import tpu as pltpu
from jax.experimental.pallas import tpu_sc as plsc

NC = 2
NS = 16
NW = NC * NS
LANES = 16
CHUNK = 128


def _round_up(a, m):
    return (a + m - 1) // m * m


def _sc_compiler_params():
    cp = pltpu.CompilerParams()
    if "needs_layout_passes" in pltpu.CompilerParams.__dataclass_fields__:
        cp = dataclasses.replace(cp, needs_layout_passes=False)
    return cp


def _deg_kernel(dst2d, n_pad, ch):
    mesh = plsc.VectorSubcoreMesh(core_axis_name="c", subcore_axis_name="s")

    @functools.partial(
        pl.kernel,
        out_type=jax.ShapeDtypeStruct((NW, n_pad), jnp.float32),
        mesh=mesh,
        scratch_types=[
            pltpu.VMEM((ch, CHUNK), jnp.int32),
            pltpu.VMEM((n_pad,), jnp.float32),
        ],
        compiler_params=_sc_compiler_params(),
    )
    def k(dst_hbm, out_hbm, didx, degbuf):
        w = lax.axis_index("c") * NS + lax.axis_index("s")
        zeros = jnp.zeros((LANES,), jnp.float32)
        ones = jnp.ones((LANES,), jnp.float32)

        pltpu.sync_copy(dst_hbm.at[pl.ds(w * ch, ch)], didx)

        @pl.loop(0, n_pad, step=LANES)
        def _(i):
            degbuf[pl.ds(i, LANES)] = zeros

        @pl.loop(0, ch)
        def _(c):
            for j in range(CHUNK // LANES):
                idx = didx[c, pl.ds(j * LANES, LANES)]
                plsc.addupdate_scatter(degbuf, [idx], ones)

        pltpu.sync_copy(degbuf, out_hbm.at[w])

    return k(dst2d)


def _prescale_kernel(deg_parts, x, n):
    def body(degp_ref, x_ref, xt_ref):
        degp = degp_ref[:, :n]
        ones = jnp.ones((NW, 1), jnp.float32)
        deg = lax.dot_general(
            degp, ones, (((0,), (0,)), ((), ())),
            preferred_element_type=jnp.float32,
            precision=lax.Precision.HIGHEST,
        )
        dis = lax.rsqrt(deg + 1.0)
        xt_ref[...] = x_ref[...] * dis

    return pl.pallas_call(
        body,
        out_shape=jax.ShapeDtypeStruct(x.shape, jnp.float32),
    )(deg_parts, x)


def _propagate_kernel(xt, src2d, dst2d, n, n_pad, ch, group):
    d = xt.shape[1]
    zero_copies = n_pad // NS // CHUNK
    out_rows = (n // NS) // 8 * 8
    out_rows_last = n - (NS - 1) * out_rows
    mesh = plsc.VectorSubcoreMesh(core_axis_name="c", subcore_axis_name="s")

    @functools.partial(
        pl.kernel,
        out_type=jax.ShapeDtypeStruct((NC, n, d), jnp.float32),
        mesh=mesh,
        scratch_types=[
            pltpu.VMEM((group, CHUNK), jnp.int32),
            pltpu.VMEM((group, CHUNK), jnp.int32),
            pltpu.VMEM((CHUNK, d), jnp.float32),
            pltpu.VMEM((CHUNK, d), jnp.float32),
            pltpu.VMEM_SHARED((n_pad, d), jnp.float32),
            pltpu.SemaphoreType.DMA,
            pltpu.SemaphoreType.DMA,
        ],
        compiler_params=_sc_compiler_params(),
    )
    def k(xt_hbm, src_hbm, dst_hbm, out_hbm, sidx, didx, rows_a, rows_b,
          h_sh, sem_a, sem_b):
        cid = lax.axis_index("c")
        sid = lax.axis_index("s")
        w = cid * NS + sid
        zeros = jnp.zeros((LANES,), jnp.float32)
        bufs = ((rows_a, sem_a), (rows_b, sem_b))

        @pl.loop(0, CHUNK)
        def _(r):
            for j in range(d // LANES):
                rows_a[r, pl.ds(j * LANES, LANES)] = zeros

        @pl.loop(0, zero_copies)
        def _(z):
            pltpu.sync_copy(
                rows_a, h_sh.at[pl.ds((sid * zero_copies + z) * CHUNK, CHUNK)]
            )

        plsc.subcore_barrier()

        @pl.loop(0, ch, step=group)
        def _(g):
            pltpu.sync_copy(src_hbm.at[pl.ds(w * ch + g, group)], sidx)
            pltpu.sync_copy(dst_hbm.at[pl.ds(w * ch + g, group)], didx)
            pltpu.async_copy(xt_hbm.at[sidx.at[0]], rows_a, sem_a)
            pltpu.async_copy(xt_hbm.at[sidx.at[1]], rows_b, sem_b)

            @pl.loop(0, group, step=2)
            def _(c):
                for i, (rows, sem) in enumerate(bufs):
                    pltpu.make_async_copy(
                        xt_hbm.at[pl.ds(0, CHUNK)], rows, sem
                    ).wait()
                    pltpu.sync_copy(rows, h_sh.at[didx.at[c + i]], add=True)

                    @pl.when(c + i + 2 < group)
                    def _():
                        pltpu.async_copy(
                            xt_hbm.at[sidx.at[c + i + 2]], rows, sem
                        )

        plsc.subcore_barrier()

        @pl.when(sid < NS - 1)
        def _():
            pltpu.sync_copy(
                h_sh.at[pl.ds(sid * out_rows, out_rows)],
                out_hbm.at[cid, pl.ds(sid * out_rows, out_rows)],
            )

        @pl.when(sid == NS - 1)
        def _():
            pltpu.sync_copy(
                h_sh.at[pl.ds((NS - 1) * out_rows, out_rows_last)],
                out_hbm.at[cid, pl.ds((NS - 1) * out_rows, out_rows_last)],
            )

    return k(xt, src2d, dst2d)


def _final_kernel(deg_parts, xt, s, w_mat, b_row, n):
    def body(degp_ref, xt_ref, s_ref, w_ref, b_ref, out_ref):
        degp = degp_ref[:, :n]
        ones = jnp.ones((NW, 1), jnp.float32)
        deg = lax.dot_general(
            degp, ones, (((0,), (0,)), ((), ())),
            preferred_element_type=jnp.float32,
            precision=lax.Precision.HIGHEST,
        )
        dis = lax.rsqrt(deg + 1.0)
        h = (s_ref[0] + s_ref[1] + xt_ref[...]) * dis
        y = lax.dot_general(
            h, w_ref[...], (((1,), (1,)), ((), ())),
            preferred_element_type=jnp.float32,
            precision=lax.Precision.HIGHEST,
        ) + b_ref[...]
        out_ref[...] = jnp.where(y >= 0.0, y, 0.1 * y)

    return pl.pallas_call(
        body,
        out_shape=jax.ShapeDtypeStruct(xt.shape, jnp.float32),
    )(deg_parts, xt, s, w_mat, b_row)


def kernel(x, edge_index, W, b):
    n, d = x.shape
    e = edge_index.shape[1]
    e_pad = _round_up(e, NW * CHUNK * 8)
    per_w = e_pad // NW
    ch = per_w // CHUNK
    n_pad = _round_up(n + 1, NS * CHUNK)

    src = edge_index[0]
    dst = edge_index[1]
    pad = e_pad - e
    src_p = jnp.concatenate([src, jnp.zeros((pad,), jnp.int32)])
    dst_p = jnp.concatenate([dst, jnp.full((pad,), n, jnp.int32)])
    src2d = src_p.reshape(e_pad // CHUNK, CHUNK)
    dst2d = dst_p.reshape(e_pad // CHUNK, CHUNK)

    deg_parts = _deg_kernel(dst2d, n_pad, ch)
    xt = _prescale_kernel(deg_parts, x, n)
    s = _propagate_kernel(xt, src2d, dst2d, n, n_pad, ch, group=16)
    return _final_kernel(deg_parts, xt, s, W, b.reshape(1, d), n)


# speedup vs baseline: 17.4566x; 1.0459x over previous
import dataclasses
import functools

import jax
import jax.numpy as jnp
from jax import lax
from jax.experimental import pallas as pl
from jax.experimental.pallas import tpu as pltpu
from jax.experimental.pallas import tpu_sc as plsc

NC = 2
NS = 16
NW = NC * NS
LANES = 16
CHUNK = 128


def _round_up(a, m):
    return (a + m - 1) // m * m


def _sc_compiler_params():
    cp = pltpu.CompilerParams()
    if "needs_layout_passes" in pltpu.CompilerParams.__dataclass_fields__:
        cp = dataclasses.replace(cp, needs_layout_passes=False)
    return cp


def _deg_kernel(dst2d, n_pad, ch0, ch1):
    mesh = plsc.VectorSubcoreMesh(core_axis_name="c", subcore_axis_name="s")
    ch_max = max(ch0, ch1)

    @functools.partial(
        pl.kernel,
        out_type=jax.ShapeDtypeStruct((NW, n_pad), jnp.float32),
        mesh=mesh,
        scratch_types=[
            pltpu.VMEM((ch_max, CHUNK), jnp.int32),
            pltpu.VMEM((n_pad,), jnp.float32),
        ],
        compiler_params=_sc_compiler_params(),
    )
    def k(dst_hbm, out_hbm, didx, degbuf):
        cid = lax.axis_index("c")
        sid = lax.axis_index("s")
        w = cid * NS + sid
        zeros = jnp.zeros((LANES,), jnp.float32)
        ones = jnp.ones((LANES,), jnp.float32)

        @pl.loop(0, n_pad, step=LANES)
        def _(i):
            degbuf[pl.ds(i, LANES)] = zeros

        def hist(base_chunk, nch):
            pltpu.sync_copy(
                dst_hbm.at[pl.ds(base_chunk, nch)], didx.at[pl.ds(0, nch)]
            )

            @pl.loop(0, nch)
            def _(c):
                for j in range(CHUNK // LANES):
                    idx = didx[c, pl.ds(j * LANES, LANES)]
                    plsc.addupdate_scatter(degbuf, [idx], ones)

        @pl.when(cid == 0)
        def _():
            hist(sid * ch0, ch0)

        @pl.when(cid == 1)
        def _():
            hist(NS * ch0 + sid * ch1, ch1)

        pltpu.sync_copy(degbuf, out_hbm.at[w])

    return k(dst2d)


def _prescale_kernel(deg_parts, x, n):
    def body(degp_ref, x_ref, xt_ref):
        degp = degp_ref[:, :n]
        ones = jnp.ones((NW, 1), jnp.float32)
        deg = lax.dot_general(
            degp, ones, (((0,), (0,)), ((), ())),
            preferred_element_type=jnp.float32,
            precision=lax.Precision.HIGHEST,
        )
        dis = lax.rsqrt(deg + 1.0)
        xt_ref[...] = x_ref[...] * dis

    return pl.pallas_call(
        body,
        out_shape=jax.ShapeDtypeStruct(x.shape, jnp.float32),
    )(deg_parts, x)


def _propagate_kernel(xt, src2d, dst2d, n, n_pad, ch0, ch1, group):
    d = xt.shape[1]
    zero_copies = n_pad // NS // CHUNK
    out_rows = (n // NS) // 8 * 8
    out_rows_last = n - (NS - 1) * out_rows
    mesh = plsc.VectorSubcoreMesh(core_axis_name="c", subcore_axis_name="s")

    @functools.partial(
        pl.kernel,
        out_type=jax.ShapeDtypeStruct((NC, n, d), jnp.float32),
        mesh=mesh,
        scratch_types=[
            pltpu.VMEM((group, CHUNK), jnp.int32),
            pltpu.VMEM((group, CHUNK), jnp.int32),
            pltpu.VMEM((CHUNK, d), jnp.float32),
            pltpu.VMEM((CHUNK, d), jnp.float32),
            pltpu.VMEM_SHARED((n_pad, d), jnp.float32),
            pltpu.SemaphoreType.DMA,
            pltpu.SemaphoreType.DMA,
        ],
        compiler_params=_sc_compiler_params(),
    )
    def k(xt_hbm, src_hbm, dst_hbm, out_hbm, sidx, didx, rows_a, rows_b,
          h_sh, sem_a, sem_b):
        cid = lax.axis_index("c")
        sid = lax.axis_index("s")
        zeros = jnp.zeros((LANES,), jnp.float32)
        bufs = ((rows_a, sem_a), (rows_b, sem_b))

        @pl.loop(0, CHUNK)
        def _(r):
            for j in range(d // LANES):
                rows_a[r, pl.ds(j * LANES, LANES)] = zeros

        @pl.loop(0, zero_copies)
        def _(z):
            pltpu.sync_copy(
                rows_a, h_sh.at[pl.ds((sid * zero_copies + z) * CHUNK, CHUNK)]
            )

        plsc.subcore_barrier()

        def edge_pipe(base_chunk, nch):
            @pl.loop(0, nch, step=group)
            def _(g):
                pltpu.sync_copy(src_hbm.at[pl.ds(base_chunk + g, group)], sidx)
                pltpu.sync_copy(dst_hbm.at[pl.ds(base_chunk + g, group)], didx)
                pltpu.async_copy(xt_hbm.at[sidx.at[0]], rows_a, sem_a)
                pltpu.async_copy(xt_hbm.at[sidx.at[1]], rows_b, sem_b)

                @pl.loop(0, group, step=2)
                def _(c):
                    for i, (rows, sem) in enumerate(bufs):
                        pltpu.make_async_copy(
                            xt_hbm.at[pl.ds(0, CHUNK)], rows, sem
                        ).wait()
                        pltpu.sync_copy(rows, h_sh.at[didx.at[c + i]], add=True)

                        @pl.when(c + i + 2 < group)
                        def _():
                            pltpu.async_copy(
                                xt_hbm.at[sidx.at[c + i + 2]], rows, sem
                            )

        @pl.when(cid == 0)
        def _():
            edge_pipe(sid * ch0, ch0)

        @pl.when(cid == 1)
        def _():
            edge_pipe(NS * ch0 + sid * ch1, ch1)

        plsc.subcore_barrier()

        @pl.when(sid < NS - 1)
        def _():
            pltpu.sync_copy(
                h_sh.at[pl.ds(sid * out_rows, out_rows)],
                out_hbm.at[cid, pl.ds(sid * out_rows, out_rows)],
            )

        @pl.when(sid == NS - 1)
        def _():
            pltpu.sync_copy(
                h_sh.at[pl.ds((NS - 1) * out_rows, out_rows_last)],
                out_hbm.at[cid, pl.ds((NS - 1) * out_rows, out_rows_last)],
            )

    return k(xt, src2d, dst2d)


def _final_kernel(deg_parts, xt, s, w_mat, b_row, n):
    def body(degp_ref, xt_ref, s_ref, w_ref, b_ref, out_ref):
        degp = degp_ref[:, :n]
        ones = jnp.ones((NW, 1), jnp.float32)
        deg = lax.dot_general(
            degp, ones, (((0,), (0,)), ((), ())),
            preferred_element_type=jnp.float32,
            precision=lax.Precision.HIGHEST,
        )
        dis = lax.rsqrt(deg + 1.0)
        h = (s_ref[0] + s_ref[1] + xt_ref[...]) * dis
        y = lax.dot_general(
            h, w_ref[...], (((1,), (1,)), ((), ())),
            preferred_element_type=jnp.float32,
            precision=lax.Precision.HIGHEST,
        ) + b_ref[...]
        out_ref[...] = jnp.where(y >= 0.0, y, 0.1 * y)

    return pl.pallas_call(
        body,
        out_shape=jax.ShapeDtypeStruct(xt.shape, jnp.float32),
    )(deg_parts, xt, s, w_mat, b_row)


def kernel(x, edge_index, W, b):
    n, d = x.shape
    e = edge_index.shape[1]
    e_pad = _round_up(e, NW * CHUNK * 8)
    ch_pair = e_pad // CHUNK // NS
    group = ch_pair // 4
    ch0 = 3 * group
    ch1 = ch_pair - ch0
    n_pad = _round_up(n + 1, NS * CHUNK)

    src = edge_index[0]
    dst = edge_index[1]
    pad = e_pad - e
    src_p = jnp.concatenate([src, jnp.zeros((pad,), jnp.int32)])
    dst_p = jnp.concatenate([dst, jnp.full((pad,), n, jnp.int32)])
    src2d = src_p.reshape(e_pad // CHUNK, CHUNK)
    dst2d = dst_p.reshape(e_pad // CHUNK, CHUNK)

    deg_parts = _deg_kernel(dst2d, n_pad, ch0, ch1)
    xt = _prescale_kernel(deg_parts, x, n)
    s = _propagate_kernel(xt, src2d, dst2d, n, n_pad, ch0, ch1, group)
    return _final_kernel(deg_parts, xt, s, W, b.reshape(1, d), n)


# speedup vs baseline: 42.7645x; 2.4498x over previous
import dataclasses
import functools

import jax
import jax.numpy as jnp
from jax import lax
from jax.experimental import pallas as pl
from jax.experimental.pallas import tpu as pltpu
from jax.experimental.pallas import tpu_sc as plsc

NC = 2
NS = 16
NW = NC * NS
LANES = 16
CHUNK = 128


def _round_up(a, m):
    return (a + m - 1) // m * m


def _sc_compiler_params():
    cp = pltpu.CompilerParams()
    if "needs_layout_passes" in pltpu.CompilerParams.__dataclass_fields__:
        cp = dataclasses.replace(cp, needs_layout_passes=False)
    return cp


def _deg_kernel(dst2d, n_pad, ch0, ch1):
    mesh = plsc.VectorSubcoreMesh(core_axis_name="c", subcore_axis_name="s")
    ch_max = max(ch0, ch1)

    @functools.partial(
        pl.kernel,
        out_type=jax.ShapeDtypeStruct((NW, n_pad), jnp.float32),
        mesh=mesh,
        scratch_types=[
            pltpu.VMEM((ch_max, CHUNK), jnp.int32),
            pltpu.VMEM((n_pad,), jnp.float32),
        ],
        compiler_params=_sc_compiler_params(),
    )
    def k(dst_hbm, out_hbm, didx, degbuf):
        cid = lax.axis_index("c")
        sid = lax.axis_index("s")
        w = cid * NS + sid
        zeros = jnp.zeros((LANES,), jnp.float32)
        ones = jnp.ones((LANES,), jnp.float32)

        @pl.loop(0, n_pad, step=LANES)
        def _(i):
            degbuf[pl.ds(i, LANES)] = zeros

        def hist(base_chunk, nch):
            pltpu.sync_copy(
                dst_hbm.at[pl.ds(base_chunk, nch)], didx.at[pl.ds(0, nch)]
            )

            @pl.loop(0, nch)
            def _(c):
                for j in range(CHUNK // LANES):
                    idx = didx[c, pl.ds(j * LANES, LANES)]
                    plsc.addupdate_scatter(degbuf, [idx], ones)

        @pl.when(cid == 0)
        def _():
            hist(sid * ch0, ch0)

        @pl.when(cid == 1)
        def _():
            hist(NS * ch0 + sid * ch1, ch1)

        pltpu.sync_copy(degbuf, out_hbm.at[w])

    return k(dst2d)


def _prescale_kernel(deg_parts, x, n):
    def body(degp_ref, x_ref, xt_ref):
        degp = degp_ref[:, :n]
        ones = jnp.ones((NW, 1), jnp.float32)
        deg = lax.dot_general(
            degp, ones, (((0,), (0,)), ((), ())),
            preferred_element_type=jnp.float32,
            precision=lax.Precision.HIGHEST,
        )
        dis = lax.rsqrt(deg + 1.0)
        xt_ref[...] = x_ref[...] * dis

    return pl.pallas_call(
        body,
        out_shape=jax.ShapeDtypeStruct(x.shape, jnp.float32),
    )(deg_parts, x)


def _propagate_kernel(xt, src2d, dst2d, n, n_pad, ch0, ch1, group):
    d = xt.shape[1]
    zero_copies = n_pad // NS // CHUNK
    out_rows = (n // NS) // 8 * 8
    out_rows_last = n - (NS - 1) * out_rows
    mesh = plsc.VectorSubcoreMesh(core_axis_name="c", subcore_axis_name="s")

    @functools.partial(
        pl.kernel,
        out_type=jax.ShapeDtypeStruct((NC, n, d), jnp.float32),
        mesh=mesh,
        scratch_types=[
            pltpu.VMEM((group, CHUNK), jnp.int32),
            pltpu.VMEM((group, CHUNK), jnp.int32),
            pltpu.VMEM((CHUNK, d), jnp.float32),
            pltpu.VMEM((CHUNK, d), jnp.float32),
            pltpu.VMEM_SHARED((n_pad, d), jnp.float32),
            pltpu.SemaphoreType.DMA,
            pltpu.SemaphoreType.DMA,
        ],
        compiler_params=_sc_compiler_params(),
    )
    def k(xt_hbm, src_hbm, dst_hbm, out_hbm, sidx, didx, rows_a, rows_b,
          h_sh, sem_a, sem_b):
        cid = lax.axis_index("c")
        sid = lax.axis_index("s")
        zeros = jnp.zeros((LANES,), jnp.float32)
        bufs = ((rows_a, sem_a), (rows_b, sem_b))

        @pl.loop(0, CHUNK)
        def _(r):
            for j in range(d // LANES):
                rows_a[r, pl.ds(j * LANES, LANES)] = zeros

        @pl.loop(0, zero_copies)
        def _(z):
            pltpu.sync_copy(
                rows_a, h_sh.at[pl.ds((sid * zero_copies + z) * CHUNK, CHUNK)]
            )

        plsc.subcore_barrier()

        def edge_pipe(base_chunk, nch):
            @pl.loop(0, nch, step=group)
            def _(g):
                pltpu.sync_copy(src_hbm.at[pl.ds(base_chunk + g, group)], sidx)
                pltpu.sync_copy(dst_hbm.at[pl.ds(base_chunk + g, group)], didx)
                pltpu.async_copy(xt_hbm.at[sidx.at[0]], rows_a, sem_a)
                pltpu.async_copy(xt_hbm.at[sidx.at[1]], rows_b, sem_b)

                @pl.loop(0, group, step=2)
                def _(c):
                    for i, (rows, sem) in enumerate(bufs):
                        pltpu.make_async_copy(
                            xt_hbm.at[pl.ds(0, CHUNK)], rows, sem
                        ).wait()
                        pltpu.sync_copy(rows, h_sh.at[didx.at[c + i]], add=True)

                        @pl.when(c + i + 2 < group)
                        def _():
                            pltpu.async_copy(
                                xt_hbm.at[sidx.at[c + i + 2]], rows, sem
                            )

        @pl.when(cid == 0)
        def _():
            edge_pipe(sid * ch0, ch0)

        @pl.when(cid == 1)
        def _():
            edge_pipe(NS * ch0 + sid * ch1, ch1)

        plsc.subcore_barrier()

        @pl.when(sid < NS - 1)
        def _():
            pltpu.sync_copy(
                h_sh.at[pl.ds(sid * out_rows, out_rows)],
                out_hbm.at[cid, pl.ds(sid * out_rows, out_rows)],
            )

        @pl.when(sid == NS - 1)
        def _():
            pltpu.sync_copy(
                h_sh.at[pl.ds((NS - 1) * out_rows, out_rows_last)],
                out_hbm.at[cid, pl.ds((NS - 1) * out_rows, out_rows_last)],
            )

    return k(xt, src2d, dst2d)


def _final_kernel(deg_parts, xt, s, w_mat, b_row, n):
    def body(degp_ref, xt_ref, s_ref, w_ref, b_ref, out_ref):
        degp = degp_ref[:, :n]
        ones = jnp.ones((NW, 1), jnp.float32)
        deg = lax.dot_general(
            degp, ones, (((0,), (0,)), ((), ())),
            preferred_element_type=jnp.float32,
            precision=lax.Precision.HIGHEST,
        )
        dis = lax.rsqrt(deg + 1.0)
        h = (s_ref[0] + s_ref[1] + xt_ref[...]) * dis
        y = lax.dot_general(
            h, w_ref[...], (((1,), (1,)), ((), ())),
            preferred_element_type=jnp.float32,
            precision=lax.Precision.HIGHEST,
        ) + b_ref[...]
        out_ref[...] = jnp.where(y >= 0.0, y, 0.1 * y)

    return pl.pallas_call(
        body,
        out_shape=jax.ShapeDtypeStruct(xt.shape, jnp.float32),
    )(deg_parts, xt, s, w_mat, b_row)


def kernel(x, edge_index, W, b):
    n, d = x.shape
    e = edge_index.shape[1]
    e_pad = _round_up(e, NW * CHUNK * 8)
    ch_pair = e_pad // CHUNK // NS
    group = ch_pair // 4
    ch0 = 2 * group
    ch1 = ch_pair - ch0
    n_pad = _round_up(n + 1, NS * CHUNK)

    src = edge_index[0]
    dst = edge_index[1]
    pad = e_pad - e
    pad_i = jnp.arange(pad, dtype=jnp.int32)
    src_p = jnp.concatenate([src, pad_i % n])
    dst_p = jnp.concatenate([dst, n + pad_i % (n_pad - n)])
    src2d = src_p.reshape(e_pad // CHUNK, CHUNK)
    dst2d = dst_p.reshape(e_pad // CHUNK, CHUNK)

    deg_parts = _deg_kernel(dst2d, n_pad, ch0, ch1)
    xt = _prescale_kernel(deg_parts, x, n)
    s = _propagate_kernel(xt, src2d, dst2d, n, n_pad, ch0, ch1, group)
    return _final_kernel(deg_parts, xt, s, W, b.reshape(1, d), n)


# speedup vs baseline: 44.5822x; 1.0425x over previous
import dataclasses
import functools

import jax
import jax.numpy as jnp
from jax import lax
from jax.experimental import pallas as pl
from jax.experimental.pallas import tpu as pltpu
from jax.experimental.pallas import tpu_sc as plsc

NC = 2
NS = 16
NW = NC * NS
LANES = 16
CHUNK = 128


def _round_up(a, m):
    return (a + m - 1) // m * m


def _sc_compiler_params():
    cp = pltpu.CompilerParams()
    if "needs_layout_passes" in pltpu.CompilerParams.__dataclass_fields__:
        cp = dataclasses.replace(cp, needs_layout_passes=False)
    return cp


def _deg_kernel(dst2d, n_pad, ch0, ch1):
    mesh = plsc.VectorSubcoreMesh(core_axis_name="c", subcore_axis_name="s")
    ch_max = max(ch0, ch1)

    @functools.partial(
        pl.kernel,
        out_type=jax.ShapeDtypeStruct((NW, n_pad), jnp.float32),
        mesh=mesh,
        scratch_types=[
            pltpu.VMEM((ch_max, CHUNK), jnp.int32),
            pltpu.VMEM((n_pad,), jnp.float32),
        ],
        compiler_params=_sc_compiler_params(),
    )
    def k(dst_hbm, out_hbm, didx, degbuf):
        cid = lax.axis_index("c")
        sid = lax.axis_index("s")
        w = cid * NS + sid
        zeros = jnp.zeros((LANES,), jnp.float32)
        ones = jnp.ones((LANES,), jnp.float32)

        @pl.loop(0, n_pad, step=LANES)
        def _(i):
            degbuf[pl.ds(i, LANES)] = zeros

        def hist(base_chunk, nch):
            pltpu.sync_copy(
                dst_hbm.at[pl.ds(base_chunk, nch)], didx.at[pl.ds(0, nch)]
            )

            @pl.loop(0, nch)
            def _(c):
                for j in range(CHUNK // LANES):
                    idx = didx[c, pl.ds(j * LANES, LANES)]
                    plsc.addupdate_scatter(degbuf, [idx], ones)

        @pl.when(cid == 0)
        def _():
            hist(sid * ch0, ch0)

        @pl.when(cid == 1)
        def _():
            hist(NS * ch0 + sid * ch1, ch1)

        pltpu.sync_copy(degbuf, out_hbm.at[w])

    return k(dst2d)


def _matmul_kernel(x, w_mat, blk):
    n, d = x.shape

    def body(x_ref, w_ref, y_ref):
        y_ref[...] = lax.dot_general(
            x_ref[...], w_ref[...], (((1,), (1,)), ((), ())),
            preferred_element_type=jnp.float32,
            precision=lax.Precision.HIGHEST,
        )

    return pl.pallas_call(
        body,
        grid=(n // blk,),
        in_specs=[
            pl.BlockSpec((blk, d), lambda i: (i, 0)),
            pl.BlockSpec((d, d), lambda i: (0, 0)),
        ],
        out_specs=pl.BlockSpec((blk, d), lambda i: (i, 0)),
        out_shape=jax.ShapeDtypeStruct((n, d), jnp.float32),
    )(x, w_mat)


def _dis_kernel(deg_parts):
    n_pad = deg_parts.shape[1]

    def body(degp_ref, dis_ref):
        ones = jnp.ones((NW, 1), jnp.float32)
        deg = lax.dot_general(
            degp_ref[...], ones, (((0,), (0,)), ((), ())),
            preferred_element_type=jnp.float32,
            precision=lax.Precision.HIGHEST,
        )
        dis_ref[...] = lax.rsqrt(deg + 1.0)

    return pl.pallas_call(
        body,
        out_shape=jax.ShapeDtypeStruct((n_pad, 1), jnp.float32),
    )(deg_parts)


def _prescale_kernel(dis, y, blk):
    n, d = y.shape

    def body(dis_ref, y_ref, yt_ref):
        yt_ref[...] = y_ref[...] * dis_ref[...]

    return pl.pallas_call(
        body,
        grid=(n // blk,),
        in_specs=[
            pl.BlockSpec((blk, 1), lambda i: (i, 0)),
            pl.BlockSpec((blk, d), lambda i: (i, 0)),
        ],
        out_specs=pl.BlockSpec((blk, d), lambda i: (i, 0)),
        out_shape=jax.ShapeDtypeStruct((n, d), jnp.float32),
    )(dis, y)


def _propagate_kernel(xt, src2d, dst2d, n, n_pad, ch0, ch1, group):
    d = xt.shape[1]
    zero_copies = n_pad // NS // CHUNK
    out_rows = (n // NS) // 8 * 8
    out_rows_last = n - (NS - 1) * out_rows
    mesh = plsc.VectorSubcoreMesh(core_axis_name="c", subcore_axis_name="s")

    @functools.partial(
        pl.kernel,
        out_type=jax.ShapeDtypeStruct((NC, n, d), jnp.float32),
        mesh=mesh,
        scratch_types=[
            pltpu.VMEM((group, CHUNK), jnp.int32),
            pltpu.VMEM((group, CHUNK), jnp.int32),
            pltpu.VMEM((CHUNK, d), jnp.float32),
            pltpu.VMEM((CHUNK, d), jnp.float32),
            pltpu.VMEM_SHARED((n_pad, d), jnp.float32),
            pltpu.SemaphoreType.DMA,
            pltpu.SemaphoreType.DMA,
        ],
        compiler_params=_sc_compiler_params(),
    )
    def k(xt_hbm, src_hbm, dst_hbm, out_hbm, sidx, didx, rows_a, rows_b,
          h_sh, sem_a, sem_b):
        cid = lax.axis_index("c")
        sid = lax.axis_index("s")
        zeros = jnp.zeros((LANES,), jnp.float32)
        bufs = ((rows_a, sem_a), (rows_b, sem_b))

        @pl.loop(0, CHUNK)
        def _(r):
            for j in range(d // LANES):
                rows_a[r, pl.ds(j * LANES, LANES)] = zeros

        @pl.loop(0, zero_copies)
        def _(z):
            pltpu.sync_copy(
                rows_a, h_sh.at[pl.ds((sid * zero_copies + z) * CHUNK, CHUNK)]
            )

        plsc.subcore_barrier()

        def edge_pipe(base_chunk, nch):
            @pl.loop(0, nch, step=group)
            def _(g):
                pltpu.sync_copy(src_hbm.at[pl.ds(base_chunk + g, group)], sidx)
                pltpu.sync_copy(dst_hbm.at[pl.ds(base_chunk + g, group)], didx)
                pltpu.async_copy(xt_hbm.at[sidx.at[0]], rows_a, sem_a)
                pltpu.async_copy(xt_hbm.at[sidx.at[1]], rows_b, sem_b)

                @pl.loop(0, group, step=2)
                def _(c):
                    for i, (rows, sem) in enumerate(bufs):
                        pltpu.make_async_copy(
                            xt_hbm.at[pl.ds(0, CHUNK)], rows, sem
                        ).wait()
                        pltpu.sync_copy(rows, h_sh.at[didx.at[c + i]], add=True)

                        @pl.when(c + i + 2 < group)
                        def _():
                            pltpu.async_copy(
                                xt_hbm.at[sidx.at[c + i + 2]], rows, sem
                            )

        @pl.when(cid == 0)
        def _():
            edge_pipe(sid * ch0, ch0)

        @pl.when(cid == 1)
        def _():
            edge_pipe(NS * ch0 + sid * ch1, ch1)

        plsc.subcore_barrier()

        @pl.when(sid < NS - 1)
        def _():
            pltpu.sync_copy(
                h_sh.at[pl.ds(sid * out_rows, out_rows)],
                out_hbm.at[cid, pl.ds(sid * out_rows, out_rows)],
            )

        @pl.when(sid == NS - 1)
        def _():
            pltpu.sync_copy(
                h_sh.at[pl.ds((NS - 1) * out_rows, out_rows_last)],
                out_hbm.at[cid, pl.ds((NS - 1) * out_rows, out_rows_last)],
            )

    return k(xt, src2d, dst2d)


def _final_kernel(dis, yt, s, b_row, blk):
    n, d = yt.shape

    def body(dis_ref, yt_ref, s_ref, b_ref, out_ref):
        h = (s_ref[0] + s_ref[1] + yt_ref[...]) * dis_ref[...]
        z = h + b_ref[...]
        out_ref[...] = jnp.where(z >= 0.0, z, 0.1 * z)

    return pl.pallas_call(
        body,
        grid=(n // blk,),
        in_specs=[
            pl.BlockSpec((blk, 1), lambda i: (i, 0)),
            pl.BlockSpec((blk, d), lambda i: (i, 0)),
            pl.BlockSpec((NC, blk, d), lambda i: (0, i, 0)),
            pl.BlockSpec((1, d), lambda i: (0, 0)),
        ],
        out_specs=pl.BlockSpec((blk, d), lambda i: (i, 0)),
        out_shape=jax.ShapeDtypeStruct((n, d), jnp.float32),
    )(dis, yt, s, b_row)


def kernel(x, edge_index, W, b):
    n, d = x.shape
    e = edge_index.shape[1]
    e_pad = _round_up(e, NW * CHUNK * 8)
    ch_pair = e_pad // CHUNK // NS
    group = ch_pair // 4
    ch0 = 2 * group
    ch1 = ch_pair - ch0
    n_pad = _round_up(n + 1, NS * CHUNK)

    src = edge_index[0]
    dst = edge_index[1]
    pad = e_pad - e
    pad_i = jnp.arange(pad, dtype=jnp.int32)
    src_pad = pad_i if pad <= n else pad_i % n
    assert n_pad - n >= 128
    src_p = jnp.concatenate([src, src_pad])
    dst_p = jnp.concatenate([dst, n + (pad_i & 127)])
    src2d = src_p.reshape(e_pad // CHUNK, CHUNK)
    dst2d = dst_p.reshape(e_pad // CHUNK, CHUNK)

    blk = 2000
    y = _matmul_kernel(x, W, blk)
    deg_parts = _deg_kernel(dst2d, n_pad, ch0, ch1)
    dis = _dis_kernel(deg_parts)
    yt = _prescale_kernel(dis, y, blk)
    s = _propagate_kernel(yt, src2d, dst2d, n, n_pad, ch0, ch1, group)
    return _final_kernel(dis, yt, s, b.reshape(1, d), blk)
